# baseline (speedup 1.0000x reference)
import functools

import jax
import jax.numpy as jnp
from jax.experimental import pallas as pl
from jax.experimental.pallas import tpu as pltpu

_K = 8
_OHEM_RATIO = 3.0
_NEG_FILL = -1e30
_BISECT_ITERS = 40

_LVL = (
    dict(base=0, nsteps=40, nc=5, L=640, total=8 * 160 * 160),
    dict(base=40, nsteps=8, nc=1, L=800, total=8 * 80 * 80),
    dict(base=48, nsteps=8, nc=1, L=200, total=8 * 40 * 40),
)
_NSTEPS = 56
_LMAX = 800


def _softplus(x):
    return jnp.maximum(x, 0.0) + jnp.log1p(jnp.exp(-jnp.abs(x)))


def _level_body(t, lvl, cls_ref, msk_ref, rx_ref, ry_ref, gx_ref, gy_ref,
                out_ref, accm_ref, accx_ref, accy_ref, maxa_ref,
                negce_ref, sel_ref):
    base, nsteps, L, total = lvl["base"], lvl["nsteps"], lvl["L"], lvl["total"]
    step = t - base

    @pl.when(step == 0)
    def _init():
        accm_ref[...] = jnp.zeros_like(accm_ref)
        accx_ref[...] = jnp.zeros_like(accx_ref)
        accy_ref[...] = jnp.zeros_like(accy_ref)
        maxa_ref[...] = jnp.zeros_like(maxa_ref)

    c = cls_ref[0]
    m = msk_ref[0].astype(jnp.float32)
    tr = m[0]
    tcl = m[1]
    tm = m[2]
    ce_tr = _softplus((c[0] - c[1]) * (2.0 * tr - 1.0))
    ce_tcl = _softplus((c[2] - c[3]) * (2.0 * tcl - 1.0))

    pos = tr * tm
    neg = tm - pos
    negce = jnp.where(neg > 0.0, ce_tr, _NEG_FILL)
    negce_ref[step, :, :L] = negce

    accm_ref[0, :, :L] += pos
    accm_ref[1, :, :L] += ce_tr * pos
    accm_ref[2, :, :L] += neg
    accm_ref[3, :, :L] += ce_tcl * pos
    accm_ref[4, :, :L] += ce_tcl
    accm_ref[5, :, :L] += ce_tr * neg
    maxa_ref[:, :L] = jnp.maximum(maxa_ref[:, :L], jnp.maximum(negce, 0.0))

    wm = (pos * (tr + tcl) * 0.125)[None]
    dx = jnp.abs(gx_ref[0] - rx_ref[0])
    mx = jnp.minimum(dx, 1.0)
    accx_ref[:, :, :L] += (dx - mx + 0.5 * mx * mx) * wm
    dy = jnp.abs(gy_ref[0] - ry_ref[0])
    my = jnp.minimum(dy, 1.0)
    accy_ref[:, :, :L] += (dy - my + 0.5 * my * my) * wm

    @pl.when(step == nsteps - 1)
    def _finalize():
        n_pos = jnp.sum(accm_ref[0, :, :L])
        s_ce_pos = jnp.sum(accm_ref[1, :, :L])
        n_neg_all = jnp.sum(accm_ref[2, :, :L])
        s_tcl_pos = jnp.sum(accm_ref[3, :, :L])
        s_tcl_all = jnp.sum(accm_ref[4, :, :L])
        s_neg_all = jnp.sum(accm_ref[5, :, :L])
        sx = jnp.sum(accx_ref[:, :, :L])
        sy = jnp.sum(accy_ref[:, :, :L])
        maxv = jnp.max(maxa_ref[:, :L])

        has_pos = n_pos > 0.0
        n_neg = jnp.where(has_pos,
                          jnp.minimum(n_neg_all,
                                      jnp.floor(_OHEM_RATIO * n_pos)),
                          100.0)
        eff = jnp.minimum(n_neg, n_neg_all)
        need_select = eff < n_neg_all

        @pl.when(need_select)
        def _bisect():
            v = negce_ref[0:nsteps, :, :L]

            def body(_, carry):
                lo, hi = carry
                mid = 0.5 * (lo + hi)
                cnt = jnp.sum((v > mid).astype(jnp.float32))
                take_lo = cnt >= eff
                return (jnp.where(take_lo, mid, lo),
                        jnp.where(take_lo, hi, mid))

            lo, hi = jax.lax.fori_loop(0, _BISECT_ITERS, body, (0.0, maxv))
            cnt_hi = jnp.sum((v > hi).astype(jnp.float32))
            sum_hi = jnp.sum(jnp.where(v > hi, v, 0.0))
            sel_ref[0] = sum_hi + (eff - cnt_hi) * hi

        loss_neg = jnp.where(need_select, sel_ref[0], s_neg_all)
        loss_pos = jnp.where(has_pos, s_ce_pos, 0.0)
        l_tr = (loss_pos + loss_neg) / (n_pos + n_neg)

        tcl_pos = s_tcl_pos / jnp.maximum(n_pos, 1.0)
        tcl_neg = (s_tcl_all - s_tcl_pos) / jnp.maximum(float(total) - n_pos,
                                                        1.0)
        l_tcl = jnp.where(has_pos, tcl_pos + 0.5 * tcl_neg, 0.0)

        denom = jnp.maximum(n_pos * float(_K), 1.0)
        l_rx = jnp.where(has_pos, sx / denom, 0.0)
        l_ry = jnp.where(has_pos, sy / denom, 0.0)

        if base == 0:
            out_ref[0] = l_tr
            out_ref[1] = l_tcl
            out_ref[2] = l_rx
            out_ref[3] = l_ry
        else:
            out_ref[0] += l_tr
            out_ref[1] += l_tcl
            out_ref[2] += l_rx
            out_ref[3] += l_ry


def _fused_kernel(c3, m3, rx3, ry3, gx3, gy3,
                  c4, m4, rx4, ry4, gx4, gy4,
                  c5, m5, rx5, ry5, gx5, gy5,
                  out_ref, accm_ref, accx_ref, accy_ref, maxa_ref,
                  negce_ref, sel_ref):
    t = pl.program_id(0)
    scratch = (out_ref, accm_ref, accx_ref, accy_ref, maxa_ref,
               negce_ref, sel_ref)
    ops = ((c3, m3, rx3, ry3, gx3, gy3),
           (c4, m4, rx4, ry4, gx4, gy4),
           (c5, m5, rx5, ry5, gx5, gy5))
    for lvl, refs in zip(_LVL, ops):
        lo, hi = lvl["base"], lvl["base"] + lvl["nsteps"]

        @pl.when((t >= lo) & (t < hi))
        def _run(lvl=lvl, refs=refs):
            _level_body(t, lvl, *refs, *scratch)


def _level_specs(lvl):
    base, nc, L = lvl["base"], lvl["nc"], lvl["L"]
    last_i = lvl["nsteps"] // nc - 1
    last_j = nc - 1

    def im(cblk):
        def f(t):
            s = t - base
            i = jnp.clip(s // nc, 0, last_i)
            j = jnp.clip(s % nc, 0, last_j)
            j = jnp.where((t >= base) & (s // nc <= last_i), j, last_j)
            j = jnp.where(t < base, 0, j)
            i = jnp.where(t < base, 0, i)
            return (i, cblk, 0, j)
        return f

    return [
        pl.BlockSpec((1, 4, 8, L), im(0)),
        pl.BlockSpec((1, 3, 8, L), im(0)),
        pl.BlockSpec((1, _K, 8, L), im(0)),
        pl.BlockSpec((1, _K, 8, L), im(1)),
        pl.BlockSpec((1, _K, 8, L), im(0)),
        pl.BlockSpec((1, _K, 8, L), im(1)),
    ]


def _prep(cls_a, reg_a, msk_a, map_a):
    n, _, s, _ = cls_a.shape
    S8 = s * s // 8
    return (cls_a.reshape(n, 4, 8, S8), msk_a.reshape(n, 3, 8, S8),
            reg_a.reshape(n, 20, 8, S8), reg_a.reshape(n, 20, 8, S8),
            map_a.reshape(n, 20, 8, S8), map_a.reshape(n, 20, 8, S8))


def kernel(p3_cls, p3_reg, p3_mask, p3_map,
           p4_cls, p4_reg, p4_mask, p4_map,
           p5_cls, p5_reg, p5_mask, p5_map):
    specs = (_level_specs(_LVL[0]) + _level_specs(_LVL[1])
             + _level_specs(_LVL[2]))
    return pl.pallas_call(
        _fused_kernel,
        grid=(_NSTEPS,),
        in_specs=specs,
        out_specs=pl.BlockSpec(memory_space=pltpu.SMEM),
        out_shape=jax.ShapeDtypeStruct((4,), jnp.float32),
        scratch_shapes=[
            pltpu.VMEM((6, 8, _LMAX), jnp.float32),
            pltpu.VMEM((_K, 8, _LMAX), jnp.float32),
            pltpu.VMEM((_K, 8, _LMAX), jnp.float32),
            pltpu.VMEM((8, _LMAX), jnp.float32),
            pltpu.VMEM((40, 8, _LMAX), jnp.float32),
            pltpu.SMEM((1,), jnp.float32),
        ],
    )(*_prep(p3_cls, p3_reg, p3_mask, p3_map),
      *_prep(p4_cls, p4_reg, p4_mask, p4_map),
      *_prep(p5_cls, p5_reg, p5_mask, p5_map))


# speedup vs baseline: 2.7339x; 2.7339x over previous
import jax
import jax.numpy as jnp
from jax.experimental import pallas as pl
from jax.experimental.pallas import tpu as pltpu

_K = 8
_OHEM_RATIO = 3.0
_NEG_FILL = -1e30
_BISECT_ITERS = 40

_LVL = (
    dict(base=0, nsteps=40, nc=5, bh=32, s=160, total=8 * 160 * 160),
    dict(base=40, nsteps=8, nc=1, bh=80, s=80, total=8 * 80 * 80),
    dict(base=48, nsteps=8, nc=1, bh=40, s=40, total=8 * 40 * 40),
)
_NSTEPS = 56


def _softplus(x):
    return jnp.maximum(x, 0.0) + jnp.log1p(jnp.exp(-jnp.abs(x)))


def _level_body(t, lvl, cls_ref, msk_ref, rx_ref, ry_ref, gx_ref, gy_ref,
                out_ref, accm_ref, accx_ref, accy_ref, maxa_ref,
                negce_ref, sel_ref):
    base, nsteps, total = lvl["base"], lvl["nsteps"], lvl["total"]
    step = t - base

    @pl.when(step == 0)
    def _init():
        accm_ref[...] = jnp.zeros_like(accm_ref)
        accx_ref[...] = jnp.zeros_like(accx_ref)
        accy_ref[...] = jnp.zeros_like(accy_ref)
        maxa_ref[...] = jnp.zeros_like(maxa_ref)

    c = cls_ref[0]
    m = msk_ref[0].astype(jnp.float32)
    tr = m[0]
    tcl = m[1]
    tm = m[2]
    ce_tr = _softplus((c[0] - c[1]) * (2.0 * tr - 1.0))
    ce_tcl = _softplus((c[2] - c[3]) * (2.0 * tcl - 1.0))

    pos = tr * tm
    neg = tm - pos
    negce = jnp.where(neg > 0.0, ce_tr, _NEG_FILL)
    negce_ref[step] = negce

    accm_ref[0] += pos
    accm_ref[1] += ce_tr * pos
    accm_ref[2] += neg
    accm_ref[3] += ce_tcl * pos
    accm_ref[4] += ce_tcl
    accm_ref[5] += ce_tr * neg
    maxa_ref[...] = jnp.maximum(maxa_ref[...], jnp.maximum(negce, 0.0))

    wm = (pos * (tr + tcl) * 0.125)[None]
    dx = jnp.abs(gx_ref[0] - rx_ref[0])
    mx = jnp.minimum(dx, 1.0)
    accx_ref[...] += (dx - mx + 0.5 * mx * mx) * wm
    dy = jnp.abs(gy_ref[0] - ry_ref[0])
    my = jnp.minimum(dy, 1.0)
    accy_ref[...] += (dy - my + 0.5 * my * my) * wm

    @pl.when(step == nsteps - 1)
    def _finalize():
        n_pos = jnp.sum(accm_ref[0])
        s_ce_pos = jnp.sum(accm_ref[1])
        n_neg_all = jnp.sum(accm_ref[2])
        s_tcl_pos = jnp.sum(accm_ref[3])
        s_tcl_all = jnp.sum(accm_ref[4])
        s_neg_all = jnp.sum(accm_ref[5])
        sx = jnp.sum(accx_ref[...])
        sy = jnp.sum(accy_ref[...])
        maxv = jnp.max(maxa_ref[...])

        has_pos = n_pos > 0.0
        n_neg = jnp.where(has_pos,
                          jnp.minimum(n_neg_all,
                                      jnp.floor(_OHEM_RATIO * n_pos)),
                          100.0)
        eff = jnp.minimum(n_neg, n_neg_all)
        need_select = eff < n_neg_all

        @pl.when(need_select)
        def _bisect():
            v = negce_ref[...]

            def body(_, carry):
                lo, hi = carry
                mid = 0.5 * (lo + hi)
                cnt = jnp.sum((v > mid).astype(jnp.float32))
                take_lo = cnt >= eff
                return (jnp.where(take_lo, mid, lo),
                        jnp.where(take_lo, hi, mid))

            lo, hi = jax.lax.fori_loop(0, _BISECT_ITERS, body, (0.0, maxv))
            cnt_hi = jnp.sum((v > hi).astype(jnp.float32))
            sum_hi = jnp.sum(jnp.where(v > hi, v, 0.0))
            sel_ref[0] = sum_hi + (eff - cnt_hi) * hi

        loss_neg = jnp.where(need_select, sel_ref[0], s_neg_all)
        loss_pos = jnp.where(has_pos, s_ce_pos, 0.0)
        l_tr = (loss_pos + loss_neg) / (n_pos + n_neg)

        tcl_pos = s_tcl_pos / jnp.maximum(n_pos, 1.0)
        tcl_neg = (s_tcl_all - s_tcl_pos) / jnp.maximum(float(total) - n_pos,
                                                        1.0)
        l_tcl = jnp.where(has_pos, tcl_pos + 0.5 * tcl_neg, 0.0)

        denom = jnp.maximum(n_pos * float(_K), 1.0)
        l_rx = jnp.where(has_pos, sx / denom, 0.0)
        l_ry = jnp.where(has_pos, sy / denom, 0.0)

        if base == 0:
            out_ref[0] = l_tr
            out_ref[1] = l_tcl
            out_ref[2] = l_rx
            out_ref[3] = l_ry
        else:
            out_ref[0] += l_tr
            out_ref[1] += l_tcl
            out_ref[2] += l_rx
            out_ref[3] += l_ry


def _fused_kernel(c3, m3, rx3, ry3, gx3, gy3,
                  c4, m4, rx4, ry4, gx4, gy4,
                  c5, m5, rx5, ry5, gx5, gy5,
                  out_ref,
                  accm3, accx3, accy3, maxa3, negce3,
                  accm4, accx4, accy4, maxa4, negce4,
                  accm5, accx5, accy5, maxa5, negce5,
                  sel_ref):
    t = pl.program_id(0)
    ops = ((c3, m3, rx3, ry3, gx3, gy3, out_ref,
            accm3, accx3, accy3, maxa3, negce3, sel_ref),
           (c4, m4, rx4, ry4, gx4, gy4, out_ref,
            accm4, accx4, accy4, maxa4, negce4, sel_ref),
           (c5, m5, rx5, ry5, gx5, gy5, out_ref,
            accm5, accx5, accy5, maxa5, negce5, sel_ref))
    for lvl, refs in zip(_LVL, ops):
        lo, hi = lvl["base"], lvl["base"] + lvl["nsteps"]

        @pl.when((t >= lo) & (t < hi))
        def _run(lvl=lvl, refs=refs):
            _level_body(t, lvl, *refs)


def _level_specs(lvl):
    base, nc, bh, s = lvl["base"], lvl["nc"], lvl["bh"], lvl["s"]
    last_i = lvl["nsteps"] // nc - 1
    last_j = nc - 1

    def im(cblk):
        def f(t):
            sidx = t - base
            i = jnp.clip(sidx // nc, 0, last_i)
            j = jnp.clip(sidx % nc, 0, last_j)
            j = jnp.where((t >= base) & (sidx // nc <= last_i), j, last_j)
            j = jnp.where(t < base, 0, j)
            i = jnp.where(t < base, 0, i)
            return (i, cblk, j, 0)
        return f

    return [
        pl.BlockSpec((1, 4, bh, s), im(0)),
        pl.BlockSpec((1, 3, bh, s), im(0)),
        pl.BlockSpec((1, _K, bh, s), im(0)),
        pl.BlockSpec((1, _K, bh, s), im(1)),
        pl.BlockSpec((1, _K, bh, s), im(0)),
        pl.BlockSpec((1, _K, bh, s), im(1)),
    ]


def _level_scratch(lvl):
    bh, s, nsteps = lvl["bh"], lvl["s"], lvl["nsteps"]
    return [
        pltpu.VMEM((6, bh, s), jnp.float32),
        pltpu.VMEM((_K, bh, s), jnp.float32),
        pltpu.VMEM((_K, bh, s), jnp.float32),
        pltpu.VMEM((bh, s), jnp.float32),
        pltpu.VMEM((nsteps, bh, s), jnp.float32),
    ]


def kernel(p3_cls, p3_reg, p3_mask, p3_map,
           p4_cls, p4_reg, p4_mask, p4_map,
           p5_cls, p5_reg, p5_mask, p5_map):
    specs = (_level_specs(_LVL[0]) + _level_specs(_LVL[1])
             + _level_specs(_LVL[2]))
    scratch = (_level_scratch(_LVL[0]) + _level_scratch(_LVL[1])
               + _level_scratch(_LVL[2]) + [pltpu.SMEM((1,), jnp.float32)])
    return pl.pallas_call(
        _fused_kernel,
        grid=(_NSTEPS,),
        in_specs=specs,
        out_specs=pl.BlockSpec(memory_space=pltpu.SMEM),
        out_shape=jax.ShapeDtypeStruct((4,), jnp.float32),
        scratch_shapes=scratch,
    )(p3_cls, p3_mask, p3_reg, p3_reg, p3_map, p3_map,
      p4_cls, p4_mask, p4_reg, p4_reg, p4_map, p4_map,
      p5_cls, p5_mask, p5_reg, p5_reg, p5_map, p5_map)
